# baseline (device time: 97919 ns/iter reference)
import jax
import jax.numpy as jnp
from jax import lax
from jax.experimental import pallas as pl
from jax.experimental.pallas import tpu as pltpu

N_DEV = 8
S = 8


def kernel(x, w_mat):
    m, _ = x.shape
    n = w_mat.shape[1]
    mc = m // N_DEV
    nr = 2 * S
    w = n // nr
    H = N_DEV - 1

    def body(x_ref, w_ref, out_ref, sb, rs_rv, ag_rv, ssems, rsems):
        p = lax.axis_index("i")
        left = lax.rem(p + N_DEV - 1, N_DEV)
        right = lax.rem(p + 1, N_DEV)

        barrier_sem = pltpu.get_barrier_semaphore()
        for nbr in (left, right):
            pl.semaphore_signal(
                barrier_sem, inc=1,
                device_id=(nbr,), device_id_type=pl.DeviceIdType.MESH,
            )
        pl.semaphore_wait(barrier_sem, 2)

        def ck(i):
            return pl.ds(i * mc, mc)

        rings = []
        for r in range(nr):
            cw = r < S
            rings.append(dict(
                cols=slice(r * w, (r + 1) * w),
                dev=right if cw else left,
                rs_r=(lambda h, cw=cw: lax.rem(
                    (p - h - 1 + N_DEV) if cw else (p + h + 1), N_DEV)),
                ag_c=(lambda g, cw=cw: lax.rem(
                    (p - g + N_DEV) if cw else (p + g), N_DEV)),
                rs=[], ag=[],
            ))

        def rcopy(src, dst, sidx, ridx, rg):
            return pltpu.make_async_remote_copy(
                src_ref=src, dst_ref=dst,
                send_sem=ssems.at[ridx, sidx], recv_sem=rsems.at[ridx, sidx],
                device_id=(rg["dev"],), device_id_type=pl.DeviceIdType.MESH,
            )

        def pchunk(i, rg):
            return jnp.dot(
                x_ref[ck(i), :], w_ref[:, rg["cols"]],
                preferred_element_type=jnp.float32,
            )

        order = [r for pair in zip(range(S), range(S, nr)) for r in pair]

        for r in order:
            rg = rings[r]
            sb[r] = pchunk(p, rg).astype(jnp.bfloat16)
            rd = rcopy(sb.at[r], rs_rv.at[r, 0], 0, r, rg)
            rd.start()
            rg["rs"].append(rd)

        for h in range(H):
            pcs = {r: pchunk(rings[r]["rs_r"](h), rings[r]) for r in order}
            for r in order:
                rg = rings[r]
                rg["rs"][h].wait_recv()
                ri = rg["rs_r"](h)
                val = pcs[r] + rs_rv[r, h].astype(jnp.float32)
                rg["rs"][h].wait_send()
                sb[r] = val.astype(jnp.bfloat16)
                if h < H - 1:
                    rd = rcopy(sb.at[r], rs_rv.at[r, h + 1], h + 1, r, rg)
                    rd.start()
                    rg["rs"].append(rd)
                else:
                    out_ref[ck(ri), rg["cols"]] = val
                    rd = rcopy(sb.at[r], ag_rv.at[r, 0], H, r, rg)
                    rd.start()
                    rg["ag"].append(rd)

        for g in range(H):
            for r in order:
                rg = rings[r]
                rg["ag"][g].wait_recv()
                if g < H - 1:
                    rd = rcopy(ag_rv.at[r, g], ag_rv.at[r, g + 1], H + 1 + g, r, rg)
                    rd.start()
                    rg["ag"].append(rd)
                out_ref[ck(rg["ag_c"](g)), rg["cols"]] = (
                    ag_rv[r, g].astype(jnp.float32)
                )

        for rg in rings:
            for g in range(H):
                rg["ag"][g].wait_send()

    n_sems = 2 * H
    return pl.pallas_call(
        body,
        out_shape=jax.ShapeDtypeStruct((m, n), jnp.float32),
        in_specs=[
            pl.BlockSpec(memory_space=pltpu.VMEM),
            pl.BlockSpec(memory_space=pltpu.VMEM),
        ],
        out_specs=pl.BlockSpec(memory_space=pltpu.VMEM),
        scratch_shapes=[
            pltpu.VMEM((nr, mc, w), jnp.bfloat16),
            pltpu.VMEM((nr, H, mc, w), jnp.bfloat16),
            pltpu.VMEM((nr, H, mc, w), jnp.bfloat16),
            pltpu.SemaphoreType.DMA((nr, n_sems)),
            pltpu.SemaphoreType.DMA((nr, n_sems)),
        ],
        compiler_params=pltpu.CompilerParams(collective_id=0),
    )(x, w_mat)


# device time: 95816 ns/iter; 1.0219x vs baseline; 1.0219x over previous
import jax
import jax.numpy as jnp
from jax import lax
from jax.experimental import pallas as pl
from jax.experimental.pallas import tpu as pltpu

N_DEV = 8
S = 4


def kernel(x, w_mat):
    m, _ = x.shape
    n = w_mat.shape[1]
    mc = m // N_DEV
    nr = 2 * S
    w = n // nr
    H = N_DEV - 1

    def body(x_ref, w_ref, out_ref, sb, rs_rv, ag_rv, ssems, rsems):
        p = lax.axis_index("i")
        left = lax.rem(p + N_DEV - 1, N_DEV)
        right = lax.rem(p + 1, N_DEV)

        barrier_sem = pltpu.get_barrier_semaphore()
        for nbr in (left, right):
            pl.semaphore_signal(
                barrier_sem, inc=1,
                device_id=(nbr,), device_id_type=pl.DeviceIdType.MESH,
            )
        pl.semaphore_wait(barrier_sem, 2)

        def ck(i):
            return pl.ds(i * mc, mc)

        rings = []
        for r in range(nr):
            cw = r < S
            rings.append(dict(
                cols=slice(r * w, (r + 1) * w),
                dev=right if cw else left,
                rs_r=(lambda h, cw=cw: lax.rem(
                    (p - h - 1 + N_DEV) if cw else (p + h + 1), N_DEV)),
                ag_c=(lambda g, cw=cw: lax.rem(
                    (p - g + N_DEV) if cw else (p + g), N_DEV)),
                rs=[], ag=[],
            ))

        def rcopy(src, dst, sidx, ridx, rg):
            return pltpu.make_async_remote_copy(
                src_ref=src, dst_ref=dst,
                send_sem=ssems.at[ridx, sidx], recv_sem=rsems.at[ridx, sidx],
                device_id=(rg["dev"],), device_id_type=pl.DeviceIdType.MESH,
            )

        def pchunk(i, rg):
            return jnp.dot(
                x_ref[ck(i), :], w_ref[:, rg["cols"]],
                preferred_element_type=jnp.float32,
            )

        order = [r for pair in zip(range(S), range(S, nr)) for r in pair]

        for r in order:
            rg = rings[r]
            sb[r] = pchunk(p, rg).astype(jnp.bfloat16)
            rd = rcopy(sb.at[r], rs_rv.at[r, 0], 0, r, rg)
            rd.start()
            rg["rs"].append(rd)

        for h in range(H):
            pcs = {r: pchunk(rings[r]["rs_r"](h), rings[r]) for r in order}
            for r in order:
                rg = rings[r]
                rg["rs"][h].wait_recv()
                ri = rg["rs_r"](h)
                val = pcs[r] + rs_rv[r, h].astype(jnp.float32)
                rg["rs"][h].wait_send()
                sb[r] = val.astype(jnp.bfloat16)
                if h < H - 1:
                    rd = rcopy(sb.at[r], rs_rv.at[r, h + 1], h + 1, r, rg)
                    rd.start()
                    rg["rs"].append(rd)
                else:
                    out_ref[ck(ri), rg["cols"]] = val
                    rd = rcopy(sb.at[r], ag_rv.at[r, 0], H, r, rg)
                    rd.start()
                    rg["ag"].append(rd)

        for g in range(H):
            for r in order:
                rg = rings[r]
                rg["ag"][g].wait_recv()
                if g < H - 1:
                    rd = rcopy(ag_rv.at[r, g], ag_rv.at[r, g + 1], H + 1 + g, r, rg)
                    rd.start()
                    rg["ag"].append(rd)
                out_ref[ck(rg["ag_c"](g)), rg["cols"]] = (
                    ag_rv[r, g].astype(jnp.float32)
                )

        for rg in rings:
            for g in range(H):
                rg["ag"][g].wait_send()

    n_sems = 2 * H
    return pl.pallas_call(
        body,
        out_shape=jax.ShapeDtypeStruct((m, n), jnp.float32),
        in_specs=[
            pl.BlockSpec(memory_space=pltpu.VMEM),
            pl.BlockSpec(memory_space=pltpu.VMEM),
        ],
        out_specs=pl.BlockSpec(memory_space=pltpu.VMEM),
        scratch_shapes=[
            pltpu.VMEM((nr, mc, w), jnp.bfloat16),
            pltpu.VMEM((nr, H, mc, w), jnp.bfloat16),
            pltpu.VMEM((nr, H, mc, w), jnp.bfloat16),
            pltpu.SemaphoreType.DMA((nr, n_sems)),
            pltpu.SemaphoreType.DMA((nr, n_sems)),
        ],
        compiler_params=pltpu.CompilerParams(collective_id=0),
    )(x, w_mat)
